# Initial kernel scaffold; baseline (speedup 1.0000x reference)
import jax
import jax.numpy as jnp
from jax.experimental import pallas as pl


def kernel(similarity_matrix, batch_emb_om, index):
    raise NotImplementedError("write your pallas kernel here")


# speedup vs baseline: 2.6950x; 2.6950x over previous
import functools

import jax
import jax.numpy as jnp
from jax import lax
from jax.experimental import pallas as pl
from jax.experimental.pallas import tpu as pltpu
from jax.experimental.pallas import tpu_sc as plsc

B = 4096
K = 4
S = 64
D = 32
SD = S * D
TEMPERATURE = 0.2
LANES = 16

NC = 2
NS = 16
NW = NC * NS
RPW = B // NW
IPW = RPW * K
CHUNK = 8
NCHUNK = RPW // CHUNK
JV = SD // LANES


def _sc_body(sim_hbm, emb_hbm, idx_hbm, w_hbm, out_hbm,
             idx_v, gidx_v, svals_v, wvals_v, nbuf, obuf, sem0, sem1):
    wid = lax.axis_index("s") * NC + lax.axis_index("c")
    base = wid * RPW

    pltpu.sync_copy(idx_hbm.at[pl.ds(wid * IPW, IPW)], idx_v)

    lane = jnp.arange(LANES, dtype=jnp.int32)
    row_in_vreg = lane >> 2
    for t in range(IPW // LANES):
        idx_chunk = idx_v[pl.ds(t * LANES, LANES)]
        row = base + t * (LANES // K) + row_in_vreg
        gidx = row * B + idx_chunk
        gidx_v[t // 8, pl.ds((t % 8) * LANES, LANES)] = gidx

    for q in range(4):
        pltpu.async_copy(sim_hbm.at[gidx_v.at[q]],
                         svals_v.at[pl.ds(q * 128, 128)], sem0)
    for q in range(4):
        pltpu.make_async_copy(sim_hbm.at[gidx_v.at[q]],
                              svals_v.at[pl.ds(q * 128, 128)], sem0).wait()

    inv_t = jnp.float32(1.0) / jnp.float32(TEMPERATURE)
    for t in range(RPW // LANES):
        ridx0 = (t * LANES + lane) * K
        s = [plsc.load_gather(svals_v, [ridx0 + k]) * inv_t for k in range(K)]
        m = jnp.maximum(jnp.maximum(s[0], s[1]), jnp.maximum(s[2], s[3]))
        e = [jnp.exp(sk - m) for sk in s]
        den = (e[0] + e[1]) + (e[2] + e[3])
        for k in range(K):
            plsc.store_scatter(wvals_v, [ridx0 + k], e[k] / den)

    pltpu.sync_copy(wvals_v, w_hbm.at[pl.ds(wid * IPW, IPW)])

    @pl.loop(0, NCHUNK)
    def chunk_body(c):
        nidx = idx_v.at[pl.ds(c * (CHUNK * K), CHUNK * K)]
        pltpu.async_copy(emb_hbm.at[nidx], nbuf, sem1).wait()
        wvecs = [wvals_v[pl.ds(c * (CHUNK * K) + h * LANES, LANES)]
                 for h in range(CHUNK * K // LANES)]
        w = [[wvecs[(r * K + k) // LANES][(r * K + k) % LANES]
              for k in range(K)] for r in range(CHUNK)]

        @pl.loop(0, JV)
        def j_body(j):
            sl = pl.ds(j * LANES, LANES)
            for r in range(CHUNK):
                acc = w[r][0] * nbuf[r * K + 0, sl]
                for k in range(1, K):
                    acc = acc + w[r][k] * nbuf[r * K + k, sl]
                obuf[r, sl] = acc

        pltpu.sync_copy(obuf, out_hbm.at[pl.ds(base + c * CHUNK, CHUNK)])


def kernel(similarity_matrix, batch_emb_om, index):
    simf = similarity_matrix.reshape(B * B)
    emb2 = batch_emb_om.reshape(B, SD)
    idxf = index.reshape(B * K)

    mesh = plsc.VectorSubcoreMesh(core_axis_name="c", subcore_axis_name="s",
                                  num_cores=NC, num_subcores=NS)
    k = pl.kernel(
        _sc_body,
        out_type=(
            jax.ShapeDtypeStruct((B * K,), jnp.float32),
            jax.ShapeDtypeStruct((B, SD), jnp.float32),
        ),
        mesh=mesh,
        scratch_types=[
            pltpu.VMEM((IPW,), jnp.int32),
            pltpu.VMEM((4, 128), jnp.int32),
            pltpu.VMEM((IPW,), jnp.float32),
            pltpu.VMEM((IPW,), jnp.float32),
            pltpu.VMEM((CHUNK * K, SD), jnp.float32),
            pltpu.VMEM((CHUNK, SD), jnp.float32),
            pltpu.SemaphoreType.DMA,
            pltpu.SemaphoreType.DMA,
        ],
        compiler_params=pltpu.CompilerParams(needs_layout_passes=False),
    )
    w_flat, out2 = k(simf, emb2, idxf)
    return (w_flat.reshape(B, K), out2.reshape(B, S, D))


# speedup vs baseline: 3.4470x; 1.2790x over previous
import functools

import jax
import jax.numpy as jnp
from jax import lax
from jax.experimental import pallas as pl
from jax.experimental.pallas import tpu as pltpu
from jax.experimental.pallas import tpu_sc as plsc

B = 4096
K = 4
S = 64
D = 32
SD = S * D
TEMPERATURE = 0.2
LANES = 16

NC = 2
NS = 16
NW = NC * NS
RPW = B // NW
IPW = RPW * K
CHUNK = 4
CK = CHUNK * K
NCHUNK = RPW // CHUNK
JV = SD // LANES


def _sc_body(sim_hbm, emb_hbm, idx_hbm, w_hbm, out_hbm,
             idx_v, gidx_v, svals_v, wvals_v, nbuf, obuf,
             sem0, semg0, semg1, semo0, semo1):
    wid = lax.axis_index("s") * NC + lax.axis_index("c")
    base = wid * RPW

    pltpu.sync_copy(idx_hbm.at[pl.ds(wid * IPW, IPW)], idx_v)

    lane = jnp.arange(LANES, dtype=jnp.int32)
    row_in_vreg = lane >> 2
    for t in range(IPW // LANES):
        idx_chunk = idx_v[pl.ds(t * LANES, LANES)]
        row = base + t * (LANES // K) + row_in_vreg
        gidx = row * B + idx_chunk
        gidx_v[t // 8, pl.ds((t % 8) * LANES, LANES)] = gidx

    for q in range(4):
        pltpu.async_copy(sim_hbm.at[gidx_v.at[q]],
                         svals_v.at[pl.ds(q * 128, 128)], sem0)
    for q in range(4):
        pltpu.make_async_copy(sim_hbm.at[gidx_v.at[q]],
                              svals_v.at[pl.ds(q * 128, 128)], sem0).wait()

    inv_t = jnp.float32(1.0) / jnp.float32(TEMPERATURE)
    for t in range(RPW // LANES):
        ridx0 = (t * LANES + lane) * K
        s = [plsc.load_gather(svals_v, [ridx0 + k]) * inv_t for k in range(K)]
        m = jnp.maximum(jnp.maximum(s[0], s[1]), jnp.maximum(s[2], s[3]))
        e = [jnp.exp(sk - m) for sk in s]
        den = (e[0] + e[1]) + (e[2] + e[3])
        for k in range(K):
            plsc.store_scatter(wvals_v, [ridx0 + k], e[k] / den)

    pltpu.sync_copy(wvals_v, w_hbm.at[pl.ds(wid * IPW, IPW)])

    semg = (semg0, semg1)
    semo = (semo0, semo1)

    def start_gather(c, b):
        pltpu.async_copy(emb_hbm.at[idx_v.at[pl.ds(c * CK, CK)]],
                         nbuf.at[b], semg[b])

    def wait_gather(c, b):
        pltpu.make_async_copy(emb_hbm.at[idx_v.at[pl.ds(c * CK, CK)]],
                              nbuf.at[b], semg[b]).wait()

    def wait_out(b):
        pltpu.make_async_copy(obuf.at[b], out_hbm.at[pl.ds(base, CHUNK)],
                              semo[b]).wait()

    start_gather(0, 0)
    start_gather(1, 1)

    @pl.loop(0, NCHUNK // 2)
    def pair_body(i):
        for b in range(2):
            c = 2 * i + b
            wait_gather(c, b)

            @pl.when(c >= 2)
            def _():
                wait_out(b)

            wvec = wvals_v[pl.ds(c * CK, CK)]
            w = [[wvec[r * K + k] for k in range(K)] for r in range(CHUNK)]

            @plsc.parallel_loop(0, JV, unroll=4)
            def j_body(j):
                sl = pl.ds(j * LANES, LANES)
                for r in range(CHUNK):
                    acc = w[r][0] * nbuf[b, r * K + 0, sl]
                    for k in range(1, K):
                        acc = acc + w[r][k] * nbuf[b, r * K + k, sl]
                    obuf[b, r, sl] = acc

            pltpu.async_copy(obuf.at[b],
                             out_hbm.at[pl.ds(base + c * CHUNK, CHUNK)],
                             semo[b])

            @pl.when(c + 2 < NCHUNK)
            def _():
                start_gather(c + 2, b)

    wait_out(0)
    wait_out(1)


def kernel(similarity_matrix, batch_emb_om, index):
    simf = similarity_matrix.reshape(B * B)
    emb2 = batch_emb_om.reshape(B, SD)
    idxf = index.reshape(B * K)

    mesh = plsc.VectorSubcoreMesh(core_axis_name="c", subcore_axis_name="s",
                                  num_cores=NC, num_subcores=NS)
    k = pl.kernel(
        _sc_body,
        out_type=(
            jax.ShapeDtypeStruct((B * K,), jnp.float32),
            jax.ShapeDtypeStruct((B, SD), jnp.float32),
        ),
        mesh=mesh,
        scratch_types=[
            pltpu.VMEM((IPW,), jnp.int32),
            pltpu.VMEM((4, 128), jnp.int32),
            pltpu.VMEM((IPW,), jnp.float32),
            pltpu.VMEM((IPW,), jnp.float32),
            pltpu.VMEM((2, CK, SD), jnp.float32),
            pltpu.VMEM((2, CHUNK, SD), jnp.float32),
            pltpu.SemaphoreType.DMA,
            pltpu.SemaphoreType.DMA,
            pltpu.SemaphoreType.DMA,
            pltpu.SemaphoreType.DMA,
            pltpu.SemaphoreType.DMA,
        ],
        compiler_params=pltpu.CompilerParams(needs_layout_passes=False),
    )
    w_flat, out2 = k(simf, emb2, idxf)
    return (w_flat.reshape(B, K), out2.reshape(B, S, D))


# speedup vs baseline: 3.6841x; 1.0688x over previous
import functools

import jax
import jax.numpy as jnp
from jax import lax
from jax.experimental import pallas as pl
from jax.experimental.pallas import tpu as pltpu
from jax.experimental.pallas import tpu_sc as plsc

B = 4096
K = 4
S = 64
D = 32
SD = S * D
TEMPERATURE = 0.2
LANES = 16

NC = 2
NS = 16
NW = NC * NS
RPW = B // NW
IPW = RPW * K
SB = 4
NSB = RPW // SB
CHUNK = 4
CK = CHUNK * K
NCHUNK = RPW // CHUNK
JV = SD // LANES


def _sc_body(sim_hbm, emb_hbm, idx_hbm, w_hbm, out_hbm,
             idx_v, svals_v, wvals_v, simbuf, nbuf, obuf,
             sems0, sems1, semg0, semg1, semo0, semo1):
    wid = lax.axis_index("s") * NC + lax.axis_index("c")
    base = wid * RPW
    lane = jnp.arange(LANES, dtype=jnp.int32)

    pltpu.sync_copy(idx_hbm.at[pl.ds(wid * IPW, IPW)], idx_v)

    semg = (semg0, semg1)
    semo = (semo0, semo1)

    def start_gather(c, b):
        pltpu.async_copy(emb_hbm.at[idx_v.at[pl.ds(c * CK, CK)]],
                         nbuf.at[b], semg[b])

    def wait_gather(c, b):
        pltpu.make_async_copy(emb_hbm.at[idx_v.at[pl.ds(c * CK, CK)]],
                              nbuf.at[b], semg[b]).wait()

    def wait_out(b):
        pltpu.make_async_copy(obuf.at[b], out_hbm.at[pl.ds(base, CHUNK)],
                              semo[b]).wait()

    start_gather(0, 0)
    start_gather(1, 1)

    sems = (sems0, sems1)

    def start_sim(sb, b):
        pltpu.async_copy(sim_hbm.at[pl.ds(base + sb * SB, SB)],
                         simbuf.at[b], sems[b])

    def wait_sim(b):
        pltpu.make_async_copy(sim_hbm.at[pl.ds(base, SB)],
                              simbuf.at[b], sems[b]).wait()

    start_sim(0, 0)
    start_sim(1, 1)
    rows4 = lane >> 2

    @pl.loop(0, NSB // 2)
    def sim_body(i):
        for b in range(2):
            sb = 2 * i + b
            wait_sim(b)
            cols = idx_v[pl.ds(sb * LANES, LANES)]
            svals_v[pl.ds(sb * LANES, LANES)] = plsc.load_gather(
                simbuf.at[b], [rows4, cols])

            @pl.when(sb + 2 < NSB)
            def _():
                start_sim(sb + 2, b)

    inv_t = jnp.float32(1.0) / jnp.float32(TEMPERATURE)
    for t in range(RPW // LANES):
        ridx0 = (t * LANES + lane) * K
        s = [plsc.load_gather(svals_v, [ridx0 + k]) * inv_t for k in range(K)]
        m = jnp.maximum(jnp.maximum(s[0], s[1]), jnp.maximum(s[2], s[3]))
        e = [jnp.exp(sk - m) for sk in s]
        den = (e[0] + e[1]) + (e[2] + e[3])
        for k in range(K):
            plsc.store_scatter(wvals_v, [ridx0 + k], e[k] / den)

    pltpu.sync_copy(wvals_v, w_hbm.at[pl.ds(wid * IPW, IPW)])

    @pl.loop(0, NCHUNK // 2)
    def pair_body(i):
        for b in range(2):
            c = 2 * i + b
            wait_gather(c, b)

            @pl.when(c >= 2)
            def _():
                wait_out(b)

            wvec = wvals_v[pl.ds(c * CK, CK)]
            w = [[wvec[r * K + k] for k in range(K)] for r in range(CHUNK)]

            @plsc.parallel_loop(0, JV, unroll=4)
            def j_body(j):
                sl = pl.ds(j * LANES, LANES)
                for r in range(CHUNK):
                    acc = w[r][0] * nbuf[b, r * K + 0, sl]
                    for k in range(1, K):
                        acc = acc + w[r][k] * nbuf[b, r * K + k, sl]
                    obuf[b, r, sl] = acc

            pltpu.async_copy(obuf.at[b],
                             out_hbm.at[pl.ds(base + c * CHUNK, CHUNK)],
                             semo[b])

            @pl.when(c + 2 < NCHUNK)
            def _():
                start_gather(c + 2, b)

    wait_out(0)
    wait_out(1)


def kernel(similarity_matrix, batch_emb_om, index):
    emb2 = batch_emb_om.reshape(B, SD)
    idxf = index.reshape(B * K)

    mesh = plsc.VectorSubcoreMesh(core_axis_name="c", subcore_axis_name="s",
                                  num_cores=NC, num_subcores=NS)
    k = pl.kernel(
        _sc_body,
        out_type=(
            jax.ShapeDtypeStruct((B * K,), jnp.float32),
            jax.ShapeDtypeStruct((B, SD), jnp.float32),
        ),
        mesh=mesh,
        scratch_types=[
            pltpu.VMEM((IPW,), jnp.int32),
            pltpu.VMEM((IPW,), jnp.float32),
            pltpu.VMEM((IPW,), jnp.float32),
            pltpu.VMEM((2, SB, B), jnp.float32),
            pltpu.VMEM((2, CK, SD), jnp.float32),
            pltpu.VMEM((2, CHUNK, SD), jnp.float32),
            pltpu.SemaphoreType.DMA,
            pltpu.SemaphoreType.DMA,
            pltpu.SemaphoreType.DMA,
            pltpu.SemaphoreType.DMA,
            pltpu.SemaphoreType.DMA,
            pltpu.SemaphoreType.DMA,
        ],
        compiler_params=pltpu.CompilerParams(needs_layout_passes=False),
    )
    w_flat, out2 = k(similarity_matrix, emb2, idxf)
    return (w_flat.reshape(B, K), out2.reshape(B, S, D))
